# Initial kernel scaffold; baseline (speedup 1.0000x reference)
import functools

import jax
import jax.numpy as jnp
from jax import lax
from jax.experimental import pallas as pl
from jax.experimental.pallas import tpu as pltpu
from jax.experimental.pallas import tpu_sc as plsc

N = 10000
E = 160000
D_IN, D_HID, D_OUT = 256, 128, 64

NC, NS = 2, 16
NW = NC * NS
CHUNK = 128
K = 40
EP = NW * K * CHUNK
ACC_ROWS = 10240
RPT = ACC_ROWS // NS
NBUF = 4

_mesh = plsc.VectorSubcoreMesh(core_axis_name="c", subcore_axis_name="s")


def _make_agg(d):
  @functools.partial(
      pl.kernel,
      out_type=jax.ShapeDtypeStruct((NC, ACC_ROWS, d), jnp.float32),
      mesh=_mesh,
      scratch_types=[
          pltpu.VMEM((K, CHUNK), jnp.int32),
          pltpu.VMEM((K, CHUNK), jnp.int32),
          pltpu.VMEM((NBUF, CHUNK, d), jnp.float32),
          pltpu.VMEM_SHARED((ACC_ROWS, d), jnp.float32),
      ] + [pltpu.SemaphoreType.DMA] * NBUF,
  )
  def agg(hs_hbm, src_hbm, dst_hbm, zero_hbm, out_hbm, srcv, dstv, rows, acc,
          *sems):
    cid = lax.axis_index("c")
    sid = lax.axis_index("s")
    wid = sid * NC + cid
    pltpu.sync_copy(src_hbm.at[pl.ds(wid * K, K)], srcv)
    pltpu.sync_copy(dst_hbm.at[pl.ds(wid * K, K)], dstv)
    pltpu.sync_copy(zero_hbm.at[pl.ds(sid * RPT, RPT)],
                    acc.at[pl.ds(sid * RPT, RPT)])
    plsc.subcore_barrier()

    for b in range(NBUF):
      pltpu.async_copy(hs_hbm.at[srcv.at[b]], rows.at[b], sems[b])

    def body(t, carry):
      for b in range(NBUF):
        g = t * NBUF + b
        pltpu.make_async_copy(hs_hbm.at[srcv.at[g]], rows.at[b],
                              sems[b]).wait()
        pltpu.sync_copy(rows.at[b], acc.at[dstv.at[g]], add=True)

        @pl.when(g + NBUF < K)
        def _issue():
          pltpu.async_copy(hs_hbm.at[srcv.at[g + NBUF]], rows.at[b], sems[b])

      return carry

    lax.fori_loop(0, K // NBUF, body, 0)
    plsc.subcore_barrier()
    pltpu.sync_copy(acc.at[pl.ds(sid * RPT, RPT)],
                    out_hbm.at[cid, pl.ds(sid * RPT, RPT)])

  return agg


_agg_hid = _make_agg(D_HID)
_agg_out = _make_agg(D_OUT)


@functools.partial(
    pl.kernel,
    out_type=jax.ShapeDtypeStruct((NC, ACC_ROWS, 8), jnp.float32),
    mesh=_mesh,
    scratch_types=[
        pltpu.VMEM((K, CHUNK), jnp.int32),
        pltpu.VMEM((CHUNK, 8), jnp.float32),
        pltpu.VMEM_SHARED((ACC_ROWS, 8), jnp.float32),
    ],
)
def _deg(dst_hbm, ones_hbm, zero_hbm, out_hbm, dstv, ones_v, acc):
  cid = lax.axis_index("c")
  sid = lax.axis_index("s")
  wid = sid * NC + cid
  pltpu.sync_copy(dst_hbm.at[pl.ds(wid * K, K)], dstv)
  pltpu.sync_copy(ones_hbm, ones_v)
  pltpu.sync_copy(zero_hbm.at[pl.ds(sid * RPT, RPT)],
                  acc.at[pl.ds(sid * RPT, RPT)])
  plsc.subcore_barrier()

  def body(g, carry):
    pltpu.sync_copy(ones_v, acc.at[dstv.at[g]], add=True)
    return carry

  lax.fori_loop(0, K, body, 0)
  plsc.subcore_barrier()
  pltpu.sync_copy(acc.at[pl.ds(sid * RPT, RPT)],
                  out_hbm.at[cid, pl.ds(sid * RPT, RPT)])


BR = 256
_GRID = (ACC_ROWS // BR,)


def _tc1_body(x_ref, w1_ref, p0_ref, p1_ref, hs_ref, dv_ref):
  h = jnp.dot(x_ref[...], w1_ref[...], preferred_element_type=jnp.float32)
  deg = p0_ref[0, :, 0:1] + p1_ref[0, :, 0:1] + 1.0
  dinv = lax.rsqrt(deg)
  hs_ref[...] = h * dinv
  dv_ref[...] = jnp.broadcast_to(dinv, (BR, 8))


def _tc2_body(q0_ref, q1_ref, hs1_ref, dv_ref, b1_ref, w2_ref, out_ref):
  dinv = dv_ref[:, 0:1]
  z = (q0_ref[0] + q1_ref[0] + hs1_ref[...]) * dinv + b1_ref[...]
  z = jnp.maximum(z, 0.0)
  h2 = jnp.dot(z, w2_ref[...], preferred_element_type=jnp.float32)
  out_ref[...] = h2 * dinv


def _tc3_body(r0_ref, r1_ref, hs2_ref, dv_ref, b2_ref, out_ref):
  dinv = dv_ref[:, 0:1]
  out_ref[...] = (r0_ref[0] + r1_ref[0] + hs2_ref[...]) * dinv + b2_ref[...]


def kernel(X, edge_index, W1, b1, W2, b2):
  ei = edge_index.astype(jnp.int32)
  src, dst = ei[0], ei[1]
  pad = EP - E
  srcp = jnp.concatenate([src, jnp.zeros((pad,), jnp.int32)])
  dstp = jnp.concatenate([dst, jnp.full((pad,), N, jnp.int32)])
  src2d = srcp.reshape(NW * K, CHUNK)
  dst2d = dstp.reshape(NW * K, CHUNK)

  ones8 = jnp.ones((CHUNK, 8), jnp.float32)
  z8 = jnp.zeros((ACC_ROWS, 8), jnp.float32)
  zh = jnp.zeros((ACC_ROWS, D_HID), jnp.float32)
  zo = jnp.zeros((ACC_ROWS, D_OUT), jnp.float32)

  degp = _deg(dst2d, ones8, z8)

  hs1, dv = pl.pallas_call(
      _tc1_body,
      grid=_GRID,
      in_specs=[
          pl.BlockSpec((BR, D_IN), lambda i: (i, 0)),
          pl.BlockSpec((D_IN, D_HID), lambda i: (0, 0)),
          pl.BlockSpec((1, BR, 8), lambda i: (0, i, 0)),
          pl.BlockSpec((1, BR, 8), lambda i: (1, i, 0)),
      ],
      out_specs=[
          pl.BlockSpec((BR, D_HID), lambda i: (i, 0)),
          pl.BlockSpec((BR, 8), lambda i: (i, 0)),
      ],
      out_shape=[
          jax.ShapeDtypeStruct((N, D_HID), jnp.float32),
          jax.ShapeDtypeStruct((N, 8), jnp.float32),
      ],
  )(X, W1, degp, degp)

  q = _agg_hid(hs1, src2d, dst2d, zh)

  hs2 = pl.pallas_call(
      _tc2_body,
      grid=_GRID,
      in_specs=[
          pl.BlockSpec((1, BR, D_HID), lambda i: (0, i, 0)),
          pl.BlockSpec((1, BR, D_HID), lambda i: (1, i, 0)),
          pl.BlockSpec((BR, D_HID), lambda i: (i, 0)),
          pl.BlockSpec((BR, 8), lambda i: (i, 0)),
          pl.BlockSpec((1, D_HID), lambda i: (0, 0)),
          pl.BlockSpec((D_HID, D_OUT), lambda i: (0, 0)),
      ],
      out_specs=pl.BlockSpec((BR, D_OUT), lambda i: (i, 0)),
      out_shape=jax.ShapeDtypeStruct((N, D_OUT), jnp.float32),
  )(q, q, hs1, dv, b1.reshape(1, D_HID), W2)

  r = _agg_out(hs2, src2d, dst2d, zo)

  out = pl.pallas_call(
      _tc3_body,
      grid=_GRID,
      in_specs=[
          pl.BlockSpec((1, BR, D_OUT), lambda i: (0, i, 0)),
          pl.BlockSpec((1, BR, D_OUT), lambda i: (1, i, 0)),
          pl.BlockSpec((BR, D_OUT), lambda i: (i, 0)),
          pl.BlockSpec((BR, 8), lambda i: (i, 0)),
          pl.BlockSpec((1, D_OUT), lambda i: (0, 0)),
      ],
      out_specs=pl.BlockSpec((BR, D_OUT), lambda i: (i, 0)),
      out_shape=jax.ShapeDtypeStruct((N, D_OUT), jnp.float32),
  )(r, r, hs2, dv, b2.reshape(1, D_OUT))

  return out


# speedup vs baseline: 9.2672x; 9.2672x over previous
import functools

import jax
import jax.numpy as jnp
from jax import lax
from jax.experimental import pallas as pl
from jax.experimental.pallas import tpu as pltpu
from jax.experimental.pallas import tpu_sc as plsc

N = 10000
E = 160000
D_IN, D_HID, D_OUT = 256, 128, 64

NC, NS = 2, 16
NW = NC * NS
CHUNK = 128
K = 40
EP = NW * K * CHUNK
ACC_ROWS = 10240
RPT = ACC_ROWS // NS
NBUF = 2


@functools.lru_cache(maxsize=None)
def _get_mesh():
  return plsc.VectorSubcoreMesh(
      core_axis_name="c", subcore_axis_name="s", num_cores=NC, num_subcores=NS)


@functools.lru_cache(maxsize=None)
def _make_agg(d):
  @functools.partial(
      pl.kernel,
      out_type=jax.ShapeDtypeStruct((NC, ACC_ROWS, d), jnp.float32),
      mesh=_get_mesh(),
      scratch_types=[
          pltpu.VMEM((K, CHUNK), jnp.int32),
          pltpu.VMEM((K, CHUNK), jnp.int32),
          pltpu.VMEM((NBUF, CHUNK, d), jnp.float32),
          pltpu.VMEM_SHARED((ACC_ROWS, d), jnp.float32),
      ] + [pltpu.SemaphoreType.DMA] * NBUF,
      compiler_params=pltpu.CompilerParams(use_tc_tiling_on_sc=False),
  )
  def agg(hs_hbm, src_hbm, dst_hbm, zero_hbm, out_hbm, srcv, dstv, rows, acc,
          *sems):
    cid = lax.axis_index("c")
    sid = lax.axis_index("s")
    wid = sid * NC + cid
    pltpu.sync_copy(src_hbm.at[pl.ds(wid * K, K)], srcv)
    pltpu.sync_copy(dst_hbm.at[pl.ds(wid * K, K)], dstv)
    pltpu.sync_copy(zero_hbm.at[pl.ds(sid * RPT, RPT)],
                    acc.at[pl.ds(sid * RPT, RPT)])
    plsc.subcore_barrier()

    for b in range(NBUF):
      pltpu.async_copy(hs_hbm.at[srcv.at[b]], rows.at[b], sems[b])

    def body(t, carry):
      for b in range(NBUF):
        g = t * NBUF + b
        pltpu.make_async_copy(hs_hbm.at[srcv.at[g]], rows.at[b],
                              sems[b]).wait()
        pltpu.sync_copy(rows.at[b], acc.at[dstv.at[g]], add=True)

        @pl.when(g + NBUF < K)
        def _issue():
          pltpu.async_copy(hs_hbm.at[srcv.at[g + NBUF]], rows.at[b], sems[b])

      return carry

    lax.fori_loop(0, K // NBUF, body, 0)
    plsc.subcore_barrier()
    pltpu.sync_copy(acc.at[pl.ds(sid * RPT, RPT)],
                    out_hbm.at[cid, pl.ds(sid * RPT, RPT)])

  return agg


@functools.lru_cache(maxsize=None)
def _make_deg():
  @functools.partial(
      pl.kernel,
      out_type=jax.ShapeDtypeStruct((NC, ACC_ROWS, 8), jnp.float32),
      mesh=_get_mesh(),
      scratch_types=[
          pltpu.VMEM((K, CHUNK), jnp.int32),
          pltpu.VMEM((CHUNK, 8), jnp.float32),
          pltpu.VMEM_SHARED((ACC_ROWS, 8), jnp.float32),
      ],
      compiler_params=pltpu.CompilerParams(use_tc_tiling_on_sc=False),
  )
  def deg(dst_hbm, ones_hbm, zero_hbm, out_hbm, dstv, ones_v, acc):
    cid = lax.axis_index("c")
    sid = lax.axis_index("s")
    wid = sid * NC + cid
    pltpu.sync_copy(dst_hbm.at[pl.ds(wid * K, K)], dstv)
    pltpu.sync_copy(ones_hbm, ones_v)
    pltpu.sync_copy(zero_hbm.at[pl.ds(sid * RPT, RPT)],
                    acc.at[pl.ds(sid * RPT, RPT)])
    plsc.subcore_barrier()

    def body(g, carry):
      pltpu.sync_copy(ones_v, acc.at[dstv.at[g]], add=True)
      return carry

    lax.fori_loop(0, K, body, 0)
    plsc.subcore_barrier()
    pltpu.sync_copy(acc.at[pl.ds(sid * RPT, RPT)],
                    out_hbm.at[cid, pl.ds(sid * RPT, RPT)])

  return deg


BR = 256
_GRID = (ACC_ROWS // BR,)


def _tc1_body(x_ref, w1_ref, p0_ref, p1_ref, hs_ref, dv_ref):
  h = jnp.dot(x_ref[...], w1_ref[...], preferred_element_type=jnp.float32)
  deg = p0_ref[0, :, 0:1] + p1_ref[0, :, 0:1] + 1.0
  dinv = lax.rsqrt(deg)
  hs_ref[...] = h * dinv
  dv_ref[...] = jnp.broadcast_to(dinv, (BR, 8))


def _tc2_body(q0_ref, q1_ref, hs1_ref, dv_ref, b1_ref, w2_ref, out_ref):
  dinv = dv_ref[:, 0:1]
  z = (q0_ref[0] + q1_ref[0] + hs1_ref[...]) * dinv + b1_ref[...]
  z = jnp.maximum(z, 0.0)
  h2 = jnp.dot(z, w2_ref[...], preferred_element_type=jnp.float32)
  out_ref[...] = h2 * dinv


def _tc3_body(r0_ref, r1_ref, hs2_ref, dv_ref, b2_ref, out_ref):
  dinv = dv_ref[:, 0:1]
  out_ref[...] = (r0_ref[0] + r1_ref[0] + hs2_ref[...]) * dinv + b2_ref[...]


def kernel(X, edge_index, W1, b1, W2, b2):
  ei = edge_index.astype(jnp.int32)
  src, dst = ei[0], ei[1]
  pad = EP - E
  srcp = jnp.concatenate([src, jnp.zeros((pad,), jnp.int32)])
  dstp = jnp.concatenate([dst, jnp.full((pad,), N, jnp.int32)])
  src2d = srcp.reshape(NW * K, CHUNK)
  dst2d = dstp.reshape(NW * K, CHUNK)

  ones8 = jnp.ones((CHUNK, 8), jnp.float32)
  z8 = jnp.zeros((ACC_ROWS, 8), jnp.float32)
  zh = jnp.zeros((ACC_ROWS, D_HID), jnp.float32)
  zo = jnp.zeros((ACC_ROWS, D_OUT), jnp.float32)

  degp = _make_deg()(dst2d, ones8, z8)

  hs1, dv = pl.pallas_call(
      _tc1_body,
      grid=_GRID,
      in_specs=[
          pl.BlockSpec((BR, D_IN), lambda i: (i, 0)),
          pl.BlockSpec((D_IN, D_HID), lambda i: (0, 0)),
          pl.BlockSpec((1, BR, 8), lambda i: (0, i, 0)),
          pl.BlockSpec((1, BR, 8), lambda i: (1, i, 0)),
      ],
      out_specs=[
          pl.BlockSpec((BR, D_HID), lambda i: (i, 0)),
          pl.BlockSpec((BR, 8), lambda i: (i, 0)),
      ],
      out_shape=[
          jax.ShapeDtypeStruct((N, D_HID), jnp.float32),
          jax.ShapeDtypeStruct((N, 8), jnp.float32),
      ],
  )(X, W1, degp, degp)

  q = _make_agg(D_HID)(hs1, src2d, dst2d, zh)

  hs2 = pl.pallas_call(
      _tc2_body,
      grid=_GRID,
      in_specs=[
          pl.BlockSpec((1, BR, D_HID), lambda i: (0, i, 0)),
          pl.BlockSpec((1, BR, D_HID), lambda i: (1, i, 0)),
          pl.BlockSpec((BR, D_HID), lambda i: (i, 0)),
          pl.BlockSpec((BR, 8), lambda i: (i, 0)),
          pl.BlockSpec((1, D_HID), lambda i: (0, 0)),
          pl.BlockSpec((D_HID, D_OUT), lambda i: (0, 0)),
      ],
      out_specs=pl.BlockSpec((BR, D_OUT), lambda i: (i, 0)),
      out_shape=jax.ShapeDtypeStruct((N, D_OUT), jnp.float32),
  )(q, q, hs1, dv, b1.reshape(1, D_HID), W2)

  r = _make_agg(D_OUT)(hs2, src2d, dst2d, zo)

  out = pl.pallas_call(
      _tc3_body,
      grid=_GRID,
      in_specs=[
          pl.BlockSpec((1, BR, D_OUT), lambda i: (0, i, 0)),
          pl.BlockSpec((1, BR, D_OUT), lambda i: (1, i, 0)),
          pl.BlockSpec((BR, D_OUT), lambda i: (i, 0)),
          pl.BlockSpec((BR, 8), lambda i: (i, 0)),
          pl.BlockSpec((1, D_OUT), lambda i: (0, 0)),
      ],
      out_specs=pl.BlockSpec((BR, D_OUT), lambda i: (i, 0)),
      out_shape=jax.ShapeDtypeStruct((N, D_OUT), jnp.float32),
  )(r, r, hs2, dv, b2.reshape(1, D_OUT))

  return out


# speedup vs baseline: 10.1218x; 1.0922x over previous
import functools

import jax
import jax.numpy as jnp
from jax import lax
from jax.experimental import pallas as pl
from jax.experimental.pallas import tpu as pltpu
from jax.experimental.pallas import tpu_sc as plsc

N = 10000
E = 160000
D_IN, D_HID, D_OUT = 256, 128, 64

NC, NS = 2, 16
NW = NC * NS
CHUNK = 128
KT = 80
ROWS2D = 1344
EP = ROWS2D * CHUNK
ACC_ROWS = 10048
RPT = ACC_ROWS // NS
NBUF = 2


@functools.lru_cache(maxsize=None)
def _get_mesh():
  return plsc.VectorSubcoreMesh(
      core_axis_name="c", subcore_axis_name="s", num_cores=NC, num_subcores=NS)


@functools.lru_cache(maxsize=None)
def _make_agg(d, k0):
  k1 = KT - k0

  @functools.partial(
      pl.kernel,
      out_type=jax.ShapeDtypeStruct((NC, ACC_ROWS, d), jnp.float32),
      mesh=_get_mesh(),
      scratch_types=[
          pltpu.VMEM((k0, CHUNK), jnp.int32),
          pltpu.VMEM((k0, CHUNK), jnp.int32),
          pltpu.VMEM((NBUF, CHUNK, d), jnp.float32),
          pltpu.VMEM_SHARED((ACC_ROWS, d), jnp.float32),
      ] + [pltpu.SemaphoreType.DMA] * NBUF,
      compiler_params=pltpu.CompilerParams(use_tc_tiling_on_sc=False),
  )
  def agg(hs_hbm, src_hbm, dst_hbm, zero_hbm, out_hbm, srcv, dstv, rows, acc,
          *sems):
    cid = lax.axis_index("c")
    sid = lax.axis_index("s")
    base = (1 - cid) * (sid * k0) + cid * (NS * k0 + sid * k1)
    k = k0 - (k0 - k1) * cid
    pltpu.sync_copy(src_hbm.at[pl.ds(base, k0)], srcv)
    pltpu.sync_copy(dst_hbm.at[pl.ds(base, k0)], dstv)
    pltpu.sync_copy(zero_hbm.at[pl.ds(sid * RPT, RPT)],
                    acc.at[pl.ds(sid * RPT, RPT)])
    plsc.subcore_barrier()

    for b in range(NBUF):
      pltpu.async_copy(hs_hbm.at[srcv.at[b]], rows.at[b], sems[b])

    def body(t, carry):
      for b in range(NBUF):
        g = t * NBUF + b
        pltpu.make_async_copy(hs_hbm.at[srcv.at[g]], rows.at[b],
                              sems[b]).wait()
        pltpu.sync_copy(rows.at[b], acc.at[dstv.at[g]], add=True)

        @pl.when(g + NBUF < k)
        def _issue():
          pltpu.async_copy(hs_hbm.at[srcv.at[g + NBUF]], rows.at[b], sems[b])

      return carry

    lax.fori_loop(0, k // NBUF, body, 0)
    plsc.subcore_barrier()
    pltpu.sync_copy(acc.at[pl.ds(sid * RPT, RPT)],
                    out_hbm.at[cid, pl.ds(sid * RPT, RPT)])

  return agg


@functools.lru_cache(maxsize=None)
def _make_deg():
  @functools.partial(
      pl.kernel,
      out_type=jax.ShapeDtypeStruct((NC, ACC_ROWS, 8), jnp.float32),
      mesh=_get_mesh(),
      scratch_types=[
          pltpu.VMEM((KT // 2, CHUNK), jnp.int32),
          pltpu.VMEM((CHUNK, 8), jnp.float32),
          pltpu.VMEM_SHARED((ACC_ROWS, 8), jnp.float32),
      ],
      compiler_params=pltpu.CompilerParams(use_tc_tiling_on_sc=False),
  )
  def deg(dst_hbm, ones_hbm, zero_hbm, out_hbm, dstv, ones_v, acc):
    cid = lax.axis_index("c")
    sid = lax.axis_index("s")
    wid = sid * NC + cid
    kh = KT // 2
    pltpu.sync_copy(dst_hbm.at[pl.ds(wid * kh, kh)], dstv)
    pltpu.sync_copy(ones_hbm, ones_v)
    pltpu.sync_copy(zero_hbm.at[pl.ds(sid * RPT, RPT)],
                    acc.at[pl.ds(sid * RPT, RPT)])
    plsc.subcore_barrier()

    def body(g, carry):
      pltpu.sync_copy(ones_v, acc.at[dstv.at[g]], add=True)
      return carry

    lax.fori_loop(0, kh, body, 0)
    plsc.subcore_barrier()
    pltpu.sync_copy(acc.at[pl.ds(sid * RPT, RPT)],
                    out_hbm.at[cid, pl.ds(sid * RPT, RPT)])

  return deg


BR = 1000
_GRID = (N // BR,)


def _dinv(p0_ref, p1_ref):
  deg = p0_ref[0, :, 0:1] + p1_ref[0, :, 0:1] + 1.0
  return lax.rsqrt(deg)


def _tc1_body(x_ref, w1_ref, p0_ref, p1_ref, hs_ref):
  h = jnp.dot(x_ref[...], w1_ref[...], preferred_element_type=jnp.float32)
  hs_ref[...] = h * _dinv(p0_ref, p1_ref)


def _tc2_body(q0_ref, q1_ref, hs1_ref, p0_ref, p1_ref, b1_ref, w2_ref,
              out_ref):
  dinv = _dinv(p0_ref, p1_ref)
  z = (q0_ref[0] + q1_ref[0] + hs1_ref[...]) * dinv + b1_ref[...]
  z = jnp.maximum(z, 0.0)
  h2 = jnp.dot(z, w2_ref[...], preferred_element_type=jnp.float32)
  out_ref[...] = h2 * dinv


def _tc3_body(r0_ref, r1_ref, hs2_ref, p0_ref, p1_ref, b2_ref, out_ref):
  dinv = _dinv(p0_ref, p1_ref)
  out_ref[...] = (r0_ref[0] + r1_ref[0] + hs2_ref[...]) * dinv + b2_ref[...]


def _pspec(minor):
  return [
      pl.BlockSpec((1, BR, minor), lambda i: (0, i, 0)),
      pl.BlockSpec((1, BR, minor), lambda i: (1, i, 0)),
  ]


def kernel(X, edge_index, W1, b1, W2, b2):
  ei = edge_index.astype(jnp.int32)
  src, dst = ei[0], ei[1]
  pad = EP - E
  srcp = jnp.concatenate([src, jnp.zeros((pad,), jnp.int32)])
  dstp = jnp.concatenate([dst, jnp.full((pad,), N, jnp.int32)])
  src2d = srcp.reshape(ROWS2D, CHUNK)
  dst2d = dstp.reshape(ROWS2D, CHUNK)

  ones8 = jnp.ones((CHUNK, 8), jnp.float32)
  z8 = jnp.zeros((ACC_ROWS, 8), jnp.float32)
  zh = jnp.zeros((ACC_ROWS, D_HID), jnp.float32)
  zo = jnp.zeros((ACC_ROWS, D_OUT), jnp.float32)

  degp = _make_deg()(dst2d, ones8, z8)

  hs1 = pl.pallas_call(
      _tc1_body,
      grid=_GRID,
      in_specs=[
          pl.BlockSpec((BR, D_IN), lambda i: (i, 0)),
          pl.BlockSpec((D_IN, D_HID), lambda i: (0, 0)),
      ] + _pspec(8),
      out_specs=pl.BlockSpec((BR, D_HID), lambda i: (i, 0)),
      out_shape=jax.ShapeDtypeStruct((N, D_HID), jnp.float32),
  )(X, W1, degp, degp)

  q = _make_agg(D_HID, 62)(hs1, src2d, dst2d, zh)

  hs2 = pl.pallas_call(
      _tc2_body,
      grid=_GRID,
      in_specs=_pspec(D_HID) + [
          pl.BlockSpec((BR, D_HID), lambda i: (i, 0)),
      ] + _pspec(8) + [
          pl.BlockSpec((1, D_HID), lambda i: (0, 0)),
          pl.BlockSpec((D_HID, D_OUT), lambda i: (0, 0)),
      ],
      out_specs=pl.BlockSpec((BR, D_OUT), lambda i: (i, 0)),
      out_shape=jax.ShapeDtypeStruct((N, D_OUT), jnp.float32),
  )(q, q, hs1, degp, degp, b1.reshape(1, D_HID), W2)

  r = _make_agg(D_OUT, 56)(hs2, src2d, dst2d, zo)

  out = pl.pallas_call(
      _tc3_body,
      grid=_GRID,
      in_specs=_pspec(D_OUT) + [
          pl.BlockSpec((BR, D_OUT), lambda i: (i, 0)),
      ] + _pspec(8) + [
          pl.BlockSpec((1, D_OUT), lambda i: (0, 0)),
      ],
      out_specs=pl.BlockSpec((BR, D_OUT), lambda i: (i, 0)),
      out_shape=jax.ShapeDtypeStruct((N, D_OUT), jnp.float32),
  )(r, r, hs2, degp, degp, b2.reshape(1, D_OUT))

  return out


# speedup vs baseline: 10.1250x; 1.0003x over previous
import functools

import jax
import jax.numpy as jnp
from jax import lax
from jax.experimental import pallas as pl
from jax.experimental.pallas import tpu as pltpu
from jax.experimental.pallas import tpu_sc as plsc

N = 10000
E = 160000
D_IN, D_HID, D_OUT = 256, 128, 64

NC, NS = 2, 16
NW = NC * NS
CHUNK = 128
KT = 80
ROWS2D = 1344
EP = ROWS2D * CHUNK
ACC_ROWS = 10048
RPT = ACC_ROWS // NS
NBUF = 2


@functools.lru_cache(maxsize=None)
def _get_mesh():
  return plsc.VectorSubcoreMesh(
      core_axis_name="c", subcore_axis_name="s", num_cores=NC, num_subcores=NS)


@functools.lru_cache(maxsize=None)
def _make_agg(d, k0):
  k1 = KT - k0

  @functools.partial(
      pl.kernel,
      out_type=jax.ShapeDtypeStruct((NC, ACC_ROWS, d), jnp.float32),
      mesh=_get_mesh(),
      scratch_types=[
          pltpu.VMEM((k0, CHUNK), jnp.int32),
          pltpu.VMEM((k0, CHUNK), jnp.int32),
          pltpu.VMEM((NBUF, CHUNK, d), jnp.float32),
          pltpu.VMEM_SHARED((ACC_ROWS, d), jnp.float32),
      ] + [pltpu.SemaphoreType.DMA] * NBUF,
      compiler_params=pltpu.CompilerParams(use_tc_tiling_on_sc=False),
  )
  def agg(hs_hbm, src_hbm, dst_hbm, zero_hbm, out_hbm, srcv, dstv, rows, acc,
          *sems):
    cid = lax.axis_index("c")
    sid = lax.axis_index("s")
    base = (1 - cid) * (sid * k0) + cid * (NS * k0 + sid * k1)
    k = k0 - (k0 - k1) * cid
    with jax.named_scope("agg_init"):
      pltpu.sync_copy(src_hbm.at[pl.ds(base, k0)], srcv)
      pltpu.sync_copy(dst_hbm.at[pl.ds(base, k0)], dstv)
      pltpu.sync_copy(zero_hbm.at[pl.ds(sid * RPT, RPT)],
                      acc.at[pl.ds(sid * RPT, RPT)])
      plsc.subcore_barrier()

    with jax.named_scope("agg_edges"):
      for b in range(NBUF):
        pltpu.async_copy(hs_hbm.at[srcv.at[b]], rows.at[b], sems[b])

      def body(t, carry):
        for b in range(NBUF):
          g = t * NBUF + b
          pltpu.make_async_copy(hs_hbm.at[srcv.at[g]], rows.at[b],
                                sems[b]).wait()
          pltpu.sync_copy(rows.at[b], acc.at[dstv.at[g]], add=True)

          @pl.when(g + NBUF < k)
          def _issue():
            pltpu.async_copy(hs_hbm.at[srcv.at[g + NBUF]], rows.at[b],
                             sems[b])

        return carry

      lax.fori_loop(0, k // NBUF, body, 0)
      plsc.subcore_barrier()

    with jax.named_scope("agg_out"):
      pltpu.sync_copy(acc.at[pl.ds(sid * RPT, RPT)],
                      out_hbm.at[cid, pl.ds(sid * RPT, RPT)])

  return agg


@functools.lru_cache(maxsize=None)
def _make_deg():
  @functools.partial(
      pl.kernel,
      out_type=jax.ShapeDtypeStruct((NC, ACC_ROWS, 8), jnp.float32),
      mesh=_get_mesh(),
      scratch_types=[
          pltpu.VMEM((KT // 2, CHUNK), jnp.int32),
          pltpu.VMEM((CHUNK, 8), jnp.float32),
          pltpu.VMEM_SHARED((ACC_ROWS, 8), jnp.float32),
      ],
      compiler_params=pltpu.CompilerParams(use_tc_tiling_on_sc=False),
  )
  def deg(dst_hbm, ones_hbm, zero_hbm, out_hbm, dstv, ones_v, acc):
    cid = lax.axis_index("c")
    sid = lax.axis_index("s")
    wid = sid * NC + cid
    kh = KT // 2
    pltpu.sync_copy(dst_hbm.at[pl.ds(wid * kh, kh)], dstv)
    pltpu.sync_copy(ones_hbm, ones_v)
    pltpu.sync_copy(zero_hbm.at[pl.ds(sid * RPT, RPT)],
                    acc.at[pl.ds(sid * RPT, RPT)])
    plsc.subcore_barrier()

    def body(g, carry):
      pltpu.sync_copy(ones_v, acc.at[dstv.at[g]], add=True)
      return carry

    lax.fori_loop(0, kh, body, 0)
    plsc.subcore_barrier()
    pltpu.sync_copy(acc.at[pl.ds(sid * RPT, RPT)],
                    out_hbm.at[cid, pl.ds(sid * RPT, RPT)])

  return deg


BR = 1000
_GRID = (N // BR,)


def _dinv(p0_ref, p1_ref):
  deg = p0_ref[0, :, 0:1] + p1_ref[0, :, 0:1] + 1.0
  return lax.rsqrt(deg)


def _tc1_body(x_ref, w1_ref, p0_ref, p1_ref, hs_ref):
  h = jnp.dot(x_ref[...], w1_ref[...], preferred_element_type=jnp.float32)
  hs_ref[...] = h * _dinv(p0_ref, p1_ref)


def _tc2_body(q0_ref, q1_ref, hs1_ref, p0_ref, p1_ref, b1_ref, w2_ref,
              out_ref):
  dinv = _dinv(p0_ref, p1_ref)
  z = (q0_ref[0] + q1_ref[0] + hs1_ref[...]) * dinv + b1_ref[...]
  z = jnp.maximum(z, 0.0)
  h2 = jnp.dot(z, w2_ref[...], preferred_element_type=jnp.float32)
  out_ref[...] = h2 * dinv


def _tc3_body(r0_ref, r1_ref, hs2_ref, p0_ref, p1_ref, b2_ref, out_ref):
  dinv = _dinv(p0_ref, p1_ref)
  out_ref[...] = (r0_ref[0] + r1_ref[0] + hs2_ref[...]) * dinv + b2_ref[...]


def _pspec(minor):
  return [
      pl.BlockSpec((1, BR, minor), lambda i: (0, i, 0)),
      pl.BlockSpec((1, BR, minor), lambda i: (1, i, 0)),
  ]


def kernel(X, edge_index, W1, b1, W2, b2):
  ei = edge_index.astype(jnp.int32)
  src, dst = ei[0], ei[1]
  pad = EP - E
  srcp = jnp.concatenate([src, jnp.zeros((pad,), jnp.int32)])
  dstp = jnp.concatenate([dst, jnp.full((pad,), N, jnp.int32)])
  src2d = srcp.reshape(ROWS2D, CHUNK)
  dst2d = dstp.reshape(ROWS2D, CHUNK)

  ones8 = jnp.ones((CHUNK, 8), jnp.float32)
  z8 = jnp.zeros((ACC_ROWS, 8), jnp.float32)
  zh = jnp.zeros((ACC_ROWS, D_HID), jnp.float32)
  zo = jnp.zeros((ACC_ROWS, D_OUT), jnp.float32)

  degp = _make_deg()(dst2d, ones8, z8)

  hs1 = pl.pallas_call(
      _tc1_body,
      grid=_GRID,
      in_specs=[
          pl.BlockSpec((BR, D_IN), lambda i: (i, 0)),
          pl.BlockSpec((D_IN, D_HID), lambda i: (0, 0)),
      ] + _pspec(8),
      out_specs=pl.BlockSpec((BR, D_HID), lambda i: (i, 0)),
      out_shape=jax.ShapeDtypeStruct((N, D_HID), jnp.float32),
  )(X, W1, degp, degp)

  q = _make_agg(D_HID, 62)(hs1, src2d, dst2d, zh)

  hs2 = pl.pallas_call(
      _tc2_body,
      grid=_GRID,
      in_specs=_pspec(D_HID) + [
          pl.BlockSpec((BR, D_HID), lambda i: (i, 0)),
      ] + _pspec(8) + [
          pl.BlockSpec((1, D_HID), lambda i: (0, 0)),
          pl.BlockSpec((D_HID, D_OUT), lambda i: (0, 0)),
      ],
      out_specs=pl.BlockSpec((BR, D_OUT), lambda i: (i, 0)),
      out_shape=jax.ShapeDtypeStruct((N, D_OUT), jnp.float32),
  )(q, q, hs1, degp, degp, b1.reshape(1, D_HID), W2)

  r = _make_agg(D_OUT, 56)(hs2, src2d, dst2d, zo)

  out = pl.pallas_call(
      _tc3_body,
      grid=_GRID,
      in_specs=_pspec(D_OUT) + [
          pl.BlockSpec((BR, D_OUT), lambda i: (i, 0)),
      ] + _pspec(8) + [
          pl.BlockSpec((1, D_OUT), lambda i: (0, 0)),
      ],
      out_specs=pl.BlockSpec((BR, D_OUT), lambda i: (i, 0)),
      out_shape=jax.ShapeDtypeStruct((N, D_OUT), jnp.float32),
  )(r, r, hs2, degp, degp, b2.reshape(1, D_OUT))

  return out


# speedup vs baseline: 10.1257x; 1.0001x over previous
import functools

import jax
import jax.numpy as jnp
from jax import lax
from jax.experimental import pallas as pl
from jax.experimental.pallas import tpu as pltpu
from jax.experimental.pallas import tpu_sc as plsc

N = 10000
E = 160000
D_IN, D_HID, D_OUT = 256, 128, 64

NC, NS = 2, 16
NW = NC * NS
CHUNK = 128
KT = 80
ROWS2D = 1344
EP = ROWS2D * CHUNK
ACC_ROWS = 10048
RPT = ACC_ROWS // NS
NBUF = 2


@functools.lru_cache(maxsize=None)
def _get_mesh():
  return plsc.VectorSubcoreMesh(
      core_axis_name="c", subcore_axis_name="s", num_cores=NC, num_subcores=NS)


@functools.lru_cache(maxsize=None)
def _make_agg(d, k0):
  k1 = KT - k0

  @functools.partial(
      pl.kernel,
      out_type=jax.ShapeDtypeStruct((NC, ACC_ROWS, d), jnp.float32),
      mesh=_get_mesh(),
      scratch_types=[
          pltpu.VMEM((k0, CHUNK), jnp.int32),
          pltpu.VMEM((k0, CHUNK), jnp.int32),
          pltpu.VMEM((NBUF, CHUNK, d), jnp.float32),
          pltpu.VMEM_SHARED((ACC_ROWS, d), jnp.float32),
      ] + [pltpu.SemaphoreType.DMA] * NBUF,
      compiler_params=pltpu.CompilerParams(use_tc_tiling_on_sc=False),
  )
  def agg(hs_hbm, src_hbm, dst_hbm, zero_hbm, out_hbm, srcv, dstv, rows, acc,
          *sems):
    cid = lax.axis_index("c")
    sid = lax.axis_index("s")
    base = (1 - cid) * (sid * k0) + cid * (NS * k0 + sid * k1)
    k = k0 - (k0 - k1) * cid
    with jax.named_scope("agg_init"):
      pltpu.sync_copy(src_hbm.at[pl.ds(base, k0)], srcv)
      pltpu.sync_copy(dst_hbm.at[pl.ds(base, k0)], dstv)
      pltpu.sync_copy(zero_hbm.at[pl.ds(sid * RPT, RPT)],
                      acc.at[pl.ds(sid * RPT, RPT)])
      plsc.subcore_barrier()

    with jax.named_scope("agg_edges"):
      for b in range(NBUF):
        pltpu.async_copy(hs_hbm.at[srcv.at[b]], rows.at[b], sems[b])

      def body(t, carry):
        for b in range(NBUF):
          g = t * NBUF + b
          with jax.named_scope("gwait"):
            pltpu.make_async_copy(hs_hbm.at[srcv.at[g]], rows.at[b],
                                  sems[b]).wait()
          with jax.named_scope("scat"):
            pltpu.sync_copy(rows.at[b], acc.at[dstv.at[g]], add=True)

          @pl.when(g + NBUF < k)
          def _issue():
            pltpu.async_copy(hs_hbm.at[srcv.at[g + NBUF]], rows.at[b],
                             sems[b])

        return carry

      lax.fori_loop(0, k // NBUF, body, 0)
      plsc.subcore_barrier()

    with jax.named_scope("agg_out"):
      pltpu.sync_copy(acc.at[pl.ds(sid * RPT, RPT)],
                      out_hbm.at[cid, pl.ds(sid * RPT, RPT)])

  return agg


@functools.lru_cache(maxsize=None)
def _make_deg():
  @functools.partial(
      pl.kernel,
      out_type=jax.ShapeDtypeStruct((NC, ACC_ROWS, 8), jnp.float32),
      mesh=_get_mesh(),
      scratch_types=[
          pltpu.VMEM((KT // 2, CHUNK), jnp.int32),
          pltpu.VMEM((CHUNK, 8), jnp.float32),
          pltpu.VMEM_SHARED((ACC_ROWS, 8), jnp.float32),
      ],
      compiler_params=pltpu.CompilerParams(use_tc_tiling_on_sc=False),
  )
  def deg(dst_hbm, ones_hbm, zero_hbm, out_hbm, dstv, ones_v, acc):
    cid = lax.axis_index("c")
    sid = lax.axis_index("s")
    wid = sid * NC + cid
    kh = KT // 2
    pltpu.sync_copy(dst_hbm.at[pl.ds(wid * kh, kh)], dstv)
    pltpu.sync_copy(ones_hbm, ones_v)
    pltpu.sync_copy(zero_hbm.at[pl.ds(sid * RPT, RPT)],
                    acc.at[pl.ds(sid * RPT, RPT)])
    plsc.subcore_barrier()

    def body(g, carry):
      pltpu.sync_copy(ones_v, acc.at[dstv.at[g]], add=True)
      return carry

    lax.fori_loop(0, kh, body, 0)
    plsc.subcore_barrier()
    pltpu.sync_copy(acc.at[pl.ds(sid * RPT, RPT)],
                    out_hbm.at[cid, pl.ds(sid * RPT, RPT)])

  return deg


BR = 1000
_GRID = (N // BR,)


def _dinv(p0_ref, p1_ref):
  deg = p0_ref[0, :, 0:1] + p1_ref[0, :, 0:1] + 1.0
  return lax.rsqrt(deg)


def _tc1_body(x_ref, w1_ref, p0_ref, p1_ref, hs_ref):
  h = jnp.dot(x_ref[...], w1_ref[...], preferred_element_type=jnp.float32)
  hs_ref[...] = h * _dinv(p0_ref, p1_ref)


def _tc2_body(q0_ref, q1_ref, hs1_ref, p0_ref, p1_ref, b1_ref, w2_ref,
              out_ref):
  dinv = _dinv(p0_ref, p1_ref)
  z = (q0_ref[0] + q1_ref[0] + hs1_ref[...]) * dinv + b1_ref[...]
  z = jnp.maximum(z, 0.0)
  h2 = jnp.dot(z, w2_ref[...], preferred_element_type=jnp.float32)
  out_ref[...] = h2 * dinv


def _tc3_body(r0_ref, r1_ref, hs2_ref, p0_ref, p1_ref, b2_ref, out_ref):
  dinv = _dinv(p0_ref, p1_ref)
  out_ref[...] = (r0_ref[0] + r1_ref[0] + hs2_ref[...]) * dinv + b2_ref[...]


def _pspec(minor):
  return [
      pl.BlockSpec((1, BR, minor), lambda i: (0, i, 0)),
      pl.BlockSpec((1, BR, minor), lambda i: (1, i, 0)),
  ]


def kernel(X, edge_index, W1, b1, W2, b2):
  ei = edge_index.astype(jnp.int32)
  src, dst = ei[0], ei[1]
  pad = EP - E
  srcp = jnp.concatenate([src, jnp.zeros((pad,), jnp.int32)])
  dstp = jnp.concatenate([dst, jnp.full((pad,), N, jnp.int32)])
  src2d = srcp.reshape(ROWS2D, CHUNK)
  dst2d = dstp.reshape(ROWS2D, CHUNK)

  ones8 = jnp.ones((CHUNK, 8), jnp.float32)
  z8 = jnp.zeros((ACC_ROWS, 8), jnp.float32)
  zh = jnp.zeros((ACC_ROWS, D_HID), jnp.float32)
  zo = jnp.zeros((ACC_ROWS, D_OUT), jnp.float32)

  degp = _make_deg()(dst2d, ones8, z8)

  hs1 = pl.pallas_call(
      _tc1_body,
      grid=_GRID,
      in_specs=[
          pl.BlockSpec((BR, D_IN), lambda i: (i, 0)),
          pl.BlockSpec((D_IN, D_HID), lambda i: (0, 0)),
      ] + _pspec(8),
      out_specs=pl.BlockSpec((BR, D_HID), lambda i: (i, 0)),
      out_shape=jax.ShapeDtypeStruct((N, D_HID), jnp.float32),
  )(X, W1, degp, degp)

  q = _make_agg(D_HID, 62)(hs1, src2d, dst2d, zh)

  hs2 = pl.pallas_call(
      _tc2_body,
      grid=_GRID,
      in_specs=_pspec(D_HID) + [
          pl.BlockSpec((BR, D_HID), lambda i: (i, 0)),
      ] + _pspec(8) + [
          pl.BlockSpec((1, D_HID), lambda i: (0, 0)),
          pl.BlockSpec((D_HID, D_OUT), lambda i: (0, 0)),
      ],
      out_specs=pl.BlockSpec((BR, D_OUT), lambda i: (i, 0)),
      out_shape=jax.ShapeDtypeStruct((N, D_OUT), jnp.float32),
  )(q, q, hs1, degp, degp, b1.reshape(1, D_HID), W2)

  r = _make_agg(D_OUT, 56)(hs2, src2d, dst2d, zo)

  out = pl.pallas_call(
      _tc3_body,
      grid=_GRID,
      in_specs=_pspec(D_OUT) + [
          pl.BlockSpec((BR, D_OUT), lambda i: (i, 0)),
      ] + _pspec(8) + [
          pl.BlockSpec((1, D_OUT), lambda i: (0, 0)),
      ],
      out_specs=pl.BlockSpec((BR, D_OUT), lambda i: (i, 0)),
      out_shape=jax.ShapeDtypeStruct((N, D_OUT), jnp.float32),
  )(r, r, hs2, degp, degp, b2.reshape(1, D_OUT))

  return out


# speedup vs baseline: 18.1340x; 1.7909x over previous
import functools

import jax
import jax.numpy as jnp
from jax import lax
from jax.experimental import pallas as pl
from jax.experimental.pallas import tpu as pltpu
from jax.experimental.pallas import tpu_sc as plsc

N = 10000
E = 160000
D_IN, D_HID, D_OUT = 256, 128, 64
DA = 64

NC, NS = 2, 16
NW = NC * NS
CHUNK = 128
K = 40
ROWS2D = NW * K
EP = ROWS2D * CHUNK
ACC_ROWS = 10048
RPT = ACC_ROWS // NS
SPT = N // NS
NBUF = 4


@functools.lru_cache(maxsize=None)
def _get_mesh():
  return plsc.VectorSubcoreMesh(
      core_axis_name="c", subcore_axis_name="s", num_cores=NC, num_subcores=NS)


@functools.lru_cache(maxsize=None)
def _make_agg():
  @functools.partial(
      pl.kernel,
      out_type=jax.ShapeDtypeStruct((NC, ACC_ROWS, DA), jnp.float32),
      mesh=_get_mesh(),
      scratch_types=[
          pltpu.VMEM((K, CHUNK), jnp.int32),
          pltpu.VMEM((K, CHUNK), jnp.int32),
          pltpu.VMEM((NBUF, CHUNK, DA), jnp.float32),
          pltpu.VMEM_SHARED((N, DA), jnp.float32),
          pltpu.VMEM_SHARED((ACC_ROWS, DA), jnp.float32),
      ] + [pltpu.SemaphoreType.DMA] * NBUF,
      compiler_params=pltpu.CompilerParams(use_tc_tiling_on_sc=False),
  )
  def agg(hs_hbm, src_hbm, dst_hbm, zero_hbm, out_hbm, srcv, dstv, rows,
          stage, acc, *sems):
    cid = lax.axis_index("c")
    sid = lax.axis_index("s")
    wid = sid * NC + cid
    pltpu.sync_copy(src_hbm.at[pl.ds(wid * K, K)], srcv)
    pltpu.sync_copy(dst_hbm.at[pl.ds(wid * K, K)], dstv)
    pltpu.sync_copy(hs_hbm.at[pl.ds(sid * SPT, SPT)],
                    stage.at[pl.ds(sid * SPT, SPT)])
    pltpu.sync_copy(zero_hbm.at[pl.ds(sid * RPT, RPT)],
                    acc.at[pl.ds(sid * RPT, RPT)])
    plsc.subcore_barrier()

    for b in range(NBUF):
      pltpu.async_copy(stage.at[srcv.at[b]], rows.at[b], sems[b])

    def body(t, carry):
      for b in range(NBUF):
        g = t * NBUF + b
        pltpu.make_async_copy(stage.at[srcv.at[g]], rows.at[b],
                              sems[b]).wait()
        pltpu.sync_copy(rows.at[b], acc.at[dstv.at[g]], add=True)

        @pl.when(g + NBUF < K)
        def _issue():
          pltpu.async_copy(stage.at[srcv.at[g + NBUF]], rows.at[b], sems[b])

      return carry

    lax.fori_loop(0, K // NBUF, body, 0)
    plsc.subcore_barrier()
    pltpu.sync_copy(acc.at[pl.ds(sid * RPT, RPT)],
                    out_hbm.at[cid, pl.ds(sid * RPT, RPT)])

  return agg


@functools.lru_cache(maxsize=None)
def _make_deg():
  @functools.partial(
      pl.kernel,
      out_type=jax.ShapeDtypeStruct((NC, ACC_ROWS, 8), jnp.float32),
      mesh=_get_mesh(),
      scratch_types=[
          pltpu.VMEM((K, CHUNK), jnp.int32),
          pltpu.VMEM((CHUNK, 8), jnp.float32),
          pltpu.VMEM_SHARED((ACC_ROWS, 8), jnp.float32),
      ],
      compiler_params=pltpu.CompilerParams(use_tc_tiling_on_sc=False),
  )
  def deg(dst_hbm, ones_hbm, zero_hbm, out_hbm, dstv, ones_v, acc):
    cid = lax.axis_index("c")
    sid = lax.axis_index("s")
    wid = sid * NC + cid
    pltpu.sync_copy(dst_hbm.at[pl.ds(wid * K, K)], dstv)
    pltpu.sync_copy(ones_hbm, ones_v)
    pltpu.sync_copy(zero_hbm.at[pl.ds(sid * RPT, RPT)],
                    acc.at[pl.ds(sid * RPT, RPT)])
    plsc.subcore_barrier()

    def body(g, carry):
      pltpu.sync_copy(ones_v, acc.at[dstv.at[g]], add=True)
      return carry

    lax.fori_loop(0, K, body, 0)
    plsc.subcore_barrier()
    pltpu.sync_copy(acc.at[pl.ds(sid * RPT, RPT)],
                    out_hbm.at[cid, pl.ds(sid * RPT, RPT)])

  return deg


BR = 1000
_GRID = (N // BR,)


def _dinv(p0_ref, p1_ref):
  deg = p0_ref[0, :, 0:1] + p1_ref[0, :, 0:1] + 1.0
  return lax.rsqrt(deg)


def _tc1_body(x_ref, w1_ref, p0_ref, p1_ref, hsa_ref, hsb_ref):
  h = jnp.dot(x_ref[...], w1_ref[...], preferred_element_type=jnp.float32)
  hs = h * _dinv(p0_ref, p1_ref)
  hsa_ref[...] = hs[:, :DA]
  hsb_ref[...] = hs[:, DA:]


def _tc2_body(qa0_ref, qa1_ref, qb0_ref, qb1_ref, hsa_ref, hsb_ref, p0_ref,
              p1_ref, b1_ref, w2_ref, out_ref):
  dinv = _dinv(p0_ref, p1_ref)
  za = (qa0_ref[0] + qa1_ref[0] + hsa_ref[...])
  zb = (qb0_ref[0] + qb1_ref[0] + hsb_ref[...])
  z = jnp.concatenate([za, zb], axis=1) * dinv + b1_ref[...]
  z = jnp.maximum(z, 0.0)
  h2 = jnp.dot(z, w2_ref[...], preferred_element_type=jnp.float32)
  out_ref[...] = h2 * dinv


def _tc3_body(r0_ref, r1_ref, hs2_ref, p0_ref, p1_ref, b2_ref, out_ref):
  dinv = _dinv(p0_ref, p1_ref)
  out_ref[...] = (r0_ref[0] + r1_ref[0] + hs2_ref[...]) * dinv + b2_ref[...]


def _pspec(minor):
  return [
      pl.BlockSpec((1, BR, minor), lambda i: (0, i, 0)),
      pl.BlockSpec((1, BR, minor), lambda i: (1, i, 0)),
  ]


def kernel(X, edge_index, W1, b1, W2, b2):
  ei = edge_index.astype(jnp.int32)
  src, dst = ei[0], ei[1]
  pad = EP - E
  srcp = jnp.concatenate([src, jnp.zeros((pad,), jnp.int32)])
  dstp = jnp.concatenate([dst, jnp.full((pad,), N, jnp.int32)])
  src2d = srcp.reshape(ROWS2D, CHUNK)
  dst2d = dstp.reshape(ROWS2D, CHUNK)

  ones8 = jnp.ones((CHUNK, 8), jnp.float32)
  z8 = jnp.zeros((ACC_ROWS, 8), jnp.float32)
  zo = jnp.zeros((ACC_ROWS, DA), jnp.float32)

  degp = _make_deg()(dst2d, ones8, z8)

  hs1a, hs1b = pl.pallas_call(
      _tc1_body,
      grid=_GRID,
      in_specs=[
          pl.BlockSpec((BR, D_IN), lambda i: (i, 0)),
          pl.BlockSpec((D_IN, D_HID), lambda i: (0, 0)),
      ] + _pspec(8),
      out_specs=[
          pl.BlockSpec((BR, DA), lambda i: (i, 0)),
          pl.BlockSpec((BR, DA), lambda i: (i, 0)),
      ],
      out_shape=[
          jax.ShapeDtypeStruct((N, DA), jnp.float32),
          jax.ShapeDtypeStruct((N, DA), jnp.float32),
      ],
  )(X, W1, degp, degp)

  agg = _make_agg()
  qa = agg(hs1a, src2d, dst2d, zo)
  qb = agg(hs1b, src2d, dst2d, zo)

  hs2 = pl.pallas_call(
      _tc2_body,
      grid=_GRID,
      in_specs=_pspec(DA) + _pspec(DA) + [
          pl.BlockSpec((BR, DA), lambda i: (i, 0)),
          pl.BlockSpec((BR, DA), lambda i: (i, 0)),
      ] + _pspec(8) + [
          pl.BlockSpec((1, D_HID), lambda i: (0, 0)),
          pl.BlockSpec((D_HID, D_OUT), lambda i: (0, 0)),
      ],
      out_specs=pl.BlockSpec((BR, D_OUT), lambda i: (i, 0)),
      out_shape=jax.ShapeDtypeStruct((N, D_OUT), jnp.float32),
  )(qa, qa, qb, qb, hs1a, hs1b, degp, degp, b1.reshape(1, D_HID), W2)

  r = agg(hs2, src2d, dst2d, zo)

  out = pl.pallas_call(
      _tc3_body,
      grid=_GRID,
      in_specs=_pspec(D_OUT) + [
          pl.BlockSpec((BR, D_OUT), lambda i: (i, 0)),
      ] + _pspec(8) + [
          pl.BlockSpec((1, D_OUT), lambda i: (0, 0)),
      ],
      out_specs=pl.BlockSpec((BR, D_OUT), lambda i: (i, 0)),
      out_shape=jax.ShapeDtypeStruct((N, D_OUT), jnp.float32),
  )(r, r, hs2, degp, degp, b2.reshape(1, D_OUT))

  return out


# speedup vs baseline: 19.4824x; 1.0744x over previous
import functools

import jax
import jax.numpy as jnp
from jax import lax
from jax.experimental import pallas as pl
from jax.experimental.pallas import tpu as pltpu
from jax.experimental.pallas import tpu_sc as plsc

N = 10000
E = 160000
D_IN, D_HID, D_OUT = 256, 128, 64
DA = 64

NC, NS = 2, 16
NW = NC * NS
CHUNK = 128
K = 40
ROWS2D = NW * K
EP = ROWS2D * CHUNK
ACC_ROWS = 10048
RPT = ACC_ROWS // NS
SPT = N // NS
NBUF = 4


@functools.lru_cache(maxsize=None)
def _get_mesh():
  return plsc.VectorSubcoreMesh(
      core_axis_name="c", subcore_axis_name="s", num_cores=NC, num_subcores=NS)


def _edge_loop(stage, acc, srcv, dstv, rows, sems, nbuf, k):
  for b in range(nbuf):
    pltpu.async_copy(stage.at[srcv.at[b]], rows.at[b], sems[b])

  def body(t, carry):
    for b in range(nbuf):
      g = t * nbuf + b
      pltpu.make_async_copy(stage.at[srcv.at[g]], rows.at[b], sems[b]).wait()
      pltpu.sync_copy(rows.at[b], acc.at[dstv.at[g]], add=True)

      @pl.when(g + nbuf < k)
      def _issue():
        pltpu.async_copy(stage.at[srcv.at[g + nbuf]], rows.at[b], sems[b])

    return carry

  lax.fori_loop(0, k // nbuf, body, 0)


@functools.lru_cache(maxsize=None)
def _make_agg_bycol():
  k = 2 * K

  @functools.partial(
      pl.kernel,
      out_type=jax.ShapeDtypeStruct((NC, ACC_ROWS, DA), jnp.float32),
      mesh=_get_mesh(),
      scratch_types=[
          pltpu.VMEM((k, CHUNK), jnp.int32),
          pltpu.VMEM((k, CHUNK), jnp.int32),
          pltpu.VMEM((2, CHUNK, DA), jnp.float32),
          pltpu.VMEM_SHARED((N, DA), jnp.float32),
          pltpu.VMEM_SHARED((ACC_ROWS, DA), jnp.float32),
      ] + [pltpu.SemaphoreType.DMA] * 4,
      compiler_params=pltpu.CompilerParams(use_tc_tiling_on_sc=False),
  )
  def agg(hsa_hbm, hsb_hbm, src_hbm, dst_hbm, zero_hbm, out_hbm, srcv, dstv,
          rows, stage, acc, *sems):
    cid = lax.axis_index("c")
    sid = lax.axis_index("s")
    pltpu.async_copy(src_hbm.at[pl.ds(sid * k, k)], srcv, sems[0])
    pltpu.async_copy(dst_hbm.at[pl.ds(sid * k, k)], dstv, sems[1])
    pltpu.async_copy(zero_hbm.at[pl.ds(sid * RPT, RPT)],
                     acc.at[pl.ds(sid * RPT, RPT)], sems[2])
    stg = stage.at[pl.ds(sid * SPT, SPT)]

    @pl.when(cid == 0)
    def _sa():
      pltpu.async_copy(hsa_hbm.at[pl.ds(sid * SPT, SPT)], stg, sems[3])

    @pl.when(cid == 1)
    def _sb():
      pltpu.async_copy(hsb_hbm.at[pl.ds(sid * SPT, SPT)], stg, sems[3])

    pltpu.make_async_copy(src_hbm.at[pl.ds(sid * k, k)], srcv,
                          sems[0]).wait()
    pltpu.make_async_copy(dst_hbm.at[pl.ds(sid * k, k)], dstv,
                          sems[1]).wait()
    pltpu.make_async_copy(zero_hbm.at[pl.ds(sid * RPT, RPT)],
                          acc.at[pl.ds(sid * RPT, RPT)], sems[2]).wait()
    pltpu.make_async_copy(hsa_hbm.at[pl.ds(sid * SPT, SPT)], stg,
                          sems[3]).wait()
    plsc.subcore_barrier()
    _edge_loop(stage, acc, srcv, dstv, rows, sems, 2, k)
    plsc.subcore_barrier()
    pltpu.sync_copy(acc.at[pl.ds(sid * RPT, RPT)],
                    out_hbm.at[cid, pl.ds(sid * RPT, RPT)])

  return agg


@functools.lru_cache(maxsize=None)
def _make_agg_byedge():
  @functools.partial(
      pl.kernel,
      out_type=jax.ShapeDtypeStruct((NC, ACC_ROWS, DA), jnp.float32),
      mesh=_get_mesh(),
      scratch_types=[
          pltpu.VMEM((K, CHUNK), jnp.int32),
          pltpu.VMEM((K, CHUNK), jnp.int32),
          pltpu.VMEM((NBUF, CHUNK, DA), jnp.float32),
          pltpu.VMEM_SHARED((N, DA), jnp.float32),
          pltpu.VMEM_SHARED((ACC_ROWS, DA), jnp.float32),
      ] + [pltpu.SemaphoreType.DMA] * NBUF,
      compiler_params=pltpu.CompilerParams(use_tc_tiling_on_sc=False),
  )
  def agg(hs_hbm, src_hbm, dst_hbm, zero_hbm, out_hbm, srcv, dstv, rows,
          stage, acc, *sems):
    cid = lax.axis_index("c")
    sid = lax.axis_index("s")
    wid = sid * NC + cid
    pltpu.async_copy(src_hbm.at[pl.ds(wid * K, K)], srcv, sems[0])
    pltpu.async_copy(dst_hbm.at[pl.ds(wid * K, K)], dstv, sems[1])
    pltpu.async_copy(zero_hbm.at[pl.ds(sid * RPT, RPT)],
                     acc.at[pl.ds(sid * RPT, RPT)], sems[2])
    pltpu.async_copy(hs_hbm.at[pl.ds(sid * SPT, SPT)],
                     stage.at[pl.ds(sid * SPT, SPT)], sems[3])
    pltpu.make_async_copy(src_hbm.at[pl.ds(wid * K, K)], srcv,
                          sems[0]).wait()
    pltpu.make_async_copy(dst_hbm.at[pl.ds(wid * K, K)], dstv,
                          sems[1]).wait()
    pltpu.make_async_copy(zero_hbm.at[pl.ds(sid * RPT, RPT)],
                          acc.at[pl.ds(sid * RPT, RPT)], sems[2]).wait()
    pltpu.make_async_copy(hs_hbm.at[pl.ds(sid * SPT, SPT)],
                          stage.at[pl.ds(sid * SPT, SPT)], sems[3]).wait()
    plsc.subcore_barrier()
    _edge_loop(stage, acc, srcv, dstv, rows, sems, NBUF, K)
    plsc.subcore_barrier()
    pltpu.sync_copy(acc.at[pl.ds(sid * RPT, RPT)],
                    out_hbm.at[cid, pl.ds(sid * RPT, RPT)])

  return agg


@functools.lru_cache(maxsize=None)
def _make_deg():
  @functools.partial(
      pl.kernel,
      out_type=jax.ShapeDtypeStruct((NC, ACC_ROWS, 8), jnp.float32),
      mesh=_get_mesh(),
      scratch_types=[
          pltpu.VMEM((K, CHUNK), jnp.int32),
          pltpu.VMEM((CHUNK, 8), jnp.float32),
          pltpu.VMEM_SHARED((ACC_ROWS, 8), jnp.float32),
      ],
      compiler_params=pltpu.CompilerParams(use_tc_tiling_on_sc=False),
  )
  def deg(dst_hbm, ones_hbm, zero_hbm, out_hbm, dstv, ones_v, acc):
    cid = lax.axis_index("c")
    sid = lax.axis_index("s")
    wid = sid * NC + cid
    pltpu.sync_copy(dst_hbm.at[pl.ds(wid * K, K)], dstv)
    pltpu.sync_copy(ones_hbm, ones_v)
    pltpu.sync_copy(zero_hbm.at[pl.ds(sid * RPT, RPT)],
                    acc.at[pl.ds(sid * RPT, RPT)])
    plsc.subcore_barrier()

    def body(g, carry):
      pltpu.sync_copy(ones_v, acc.at[dstv.at[g]], add=True)
      return carry

    lax.fori_loop(0, K, body, 0)
    plsc.subcore_barrier()
    pltpu.sync_copy(acc.at[pl.ds(sid * RPT, RPT)],
                    out_hbm.at[cid, pl.ds(sid * RPT, RPT)])

  return deg


BR = 1000
_GRID = (N // BR,)


def _dinv(p0_ref, p1_ref):
  deg = p0_ref[0, :, 0:1] + p1_ref[0, :, 0:1] + 1.0
  return lax.rsqrt(deg)


def _tc1_body(x_ref, w1_ref, p0_ref, p1_ref, hsa_ref, hsb_ref):
  h = jnp.dot(x_ref[...], w1_ref[...], preferred_element_type=jnp.float32)
  hs = h * _dinv(p0_ref, p1_ref)
  hsa_ref[...] = hs[:, :DA]
  hsb_ref[...] = hs[:, DA:]


def _tc2_body(qa_ref, qb_ref, hsa_ref, hsb_ref, p0_ref,
              p1_ref, b1_ref, w2_ref, out_ref):
  dinv = _dinv(p0_ref, p1_ref)
  za = (qa_ref[0] + hsa_ref[...])
  zb = (qb_ref[0] + hsb_ref[...])
  z = jnp.concatenate([za, zb], axis=1) * dinv + b1_ref[...]
  z = jnp.maximum(z, 0.0)
  h2 = jnp.dot(z, w2_ref[...], preferred_element_type=jnp.float32)
  out_ref[...] = h2 * dinv


def _tc3_body(r0_ref, r1_ref, hs2_ref, p0_ref, p1_ref, b2_ref, out_ref):
  dinv = _dinv(p0_ref, p1_ref)
  out_ref[...] = (r0_ref[0] + r1_ref[0] + hs2_ref[...]) * dinv + b2_ref[...]


def _pspec(minor):
  return [
      pl.BlockSpec((1, BR, minor), lambda i: (0, i, 0)),
      pl.BlockSpec((1, BR, minor), lambda i: (1, i, 0)),
  ]


def kernel(X, edge_index, W1, b1, W2, b2):
  ei = edge_index.astype(jnp.int32)
  src, dst = ei[0], ei[1]
  pad = EP - E
  srcp = jnp.concatenate([src, jnp.zeros((pad,), jnp.int32)])
  dstp = jnp.concatenate([dst, jnp.full((pad,), N, jnp.int32)])
  src2d = srcp.reshape(ROWS2D, CHUNK)
  dst2d = dstp.reshape(ROWS2D, CHUNK)

  ones8 = jnp.ones((CHUNK, 8), jnp.float32)
  z8 = jnp.zeros((ACC_ROWS, 8), jnp.float32)
  zo = jnp.zeros((ACC_ROWS, DA), jnp.float32)

  degp = _make_deg()(dst2d, ones8, z8)

  hs1a, hs1b = pl.pallas_call(
      _tc1_body,
      grid=_GRID,
      in_specs=[
          pl.BlockSpec((BR, D_IN), lambda i: (i, 0)),
          pl.BlockSpec((D_IN, D_HID), lambda i: (0, 0)),
      ] + _pspec(8),
      out_specs=[
          pl.BlockSpec((BR, DA), lambda i: (i, 0)),
          pl.BlockSpec((BR, DA), lambda i: (i, 0)),
      ],
      out_shape=[
          jax.ShapeDtypeStruct((N, DA), jnp.float32),
          jax.ShapeDtypeStruct((N, DA), jnp.float32),
      ],
  )(X, W1, degp, degp)

  qab = _make_agg_bycol()(hs1a, hs1b, src2d, dst2d, zo)

  hs2 = pl.pallas_call(
      _tc2_body,
      grid=_GRID,
      in_specs=_pspec(DA) + [
          pl.BlockSpec((BR, DA), lambda i: (i, 0)),
          pl.BlockSpec((BR, DA), lambda i: (i, 0)),
      ] + _pspec(8) + [
          pl.BlockSpec((1, D_HID), lambda i: (0, 0)),
          pl.BlockSpec((D_HID, D_OUT), lambda i: (0, 0)),
      ],
      out_specs=pl.BlockSpec((BR, D_OUT), lambda i: (i, 0)),
      out_shape=jax.ShapeDtypeStruct((N, D_OUT), jnp.float32),
  )(qab, qab, hs1a, hs1b, degp, degp, b1.reshape(1, D_HID), W2)

  r = _make_agg_byedge()(hs2, src2d, dst2d, zo)

  out = pl.pallas_call(
      _tc3_body,
      grid=_GRID,
      in_specs=_pspec(D_OUT) + [
          pl.BlockSpec((BR, D_OUT), lambda i: (i, 0)),
      ] + _pspec(8) + [
          pl.BlockSpec((1, D_OUT), lambda i: (0, 0)),
      ],
      out_specs=pl.BlockSpec((BR, D_OUT), lambda i: (i, 0)),
      out_shape=jax.ShapeDtypeStruct((N, D_OUT), jnp.float32),
  )(r, r, hs2, degp, degp, b2.reshape(1, D_OUT))

  return out


# speedup vs baseline: 19.8986x; 1.0214x over previous
import functools

import jax
import jax.numpy as jnp
from jax import lax
from jax.experimental import pallas as pl
from jax.experimental.pallas import tpu as pltpu
from jax.experimental.pallas import tpu_sc as plsc

N = 10000
E = 160000
D_IN, D_HID, D_OUT = 256, 128, 64
DA = 64

NC, NS = 2, 16
NW = NC * NS
CHUNK = 128
K = 40
ROWS2D = NW * K
EP = ROWS2D * CHUNK
ACC_ROWS = 10048
RPT = ACC_ROWS // NS
SPT = N // NS
NBUF = 4


@functools.lru_cache(maxsize=None)
def _get_mesh():
  return plsc.VectorSubcoreMesh(
      core_axis_name="c", subcore_axis_name="s", num_cores=NC, num_subcores=NS)


def _edge_loop(stage, acc, srcv, dstv, rows, sems, nbuf, k):
  for b in range(nbuf):
    pltpu.async_copy(stage.at[srcv.at[b]], rows.at[b], sems[b])

  def body(t, carry):
    for b in range(nbuf):
      g = t * nbuf + b
      pltpu.make_async_copy(stage.at[srcv.at[g]], rows.at[b], sems[b]).wait()
      pltpu.sync_copy(rows.at[b], acc.at[dstv.at[g]], add=True)

      @pl.when(g + nbuf < k)
      def _issue():
        pltpu.async_copy(stage.at[srcv.at[g + nbuf]], rows.at[b], sems[b])

    return carry

  lax.fori_loop(0, k // nbuf, body, 0)


@functools.lru_cache(maxsize=None)
def _make_agg_bycol():
  k = 2 * K

  @functools.partial(
      pl.kernel,
      out_type=jax.ShapeDtypeStruct((NC, ACC_ROWS, DA), jnp.float32),
      mesh=_get_mesh(),
      scratch_types=[
          pltpu.VMEM((k, CHUNK), jnp.int32),
          pltpu.VMEM((k, CHUNK), jnp.int32),
          pltpu.VMEM((2, CHUNK, DA), jnp.float32),
          pltpu.VMEM_SHARED((N, DA), jnp.float32),
          pltpu.VMEM_SHARED((ACC_ROWS, DA), jnp.float32),
      ] + [pltpu.SemaphoreType.DMA] * 4,
      compiler_params=pltpu.CompilerParams(use_tc_tiling_on_sc=False),
  )
  def agg(hsa_hbm, hsb_hbm, src_hbm, dst_hbm, zero_hbm, out_hbm, srcv, dstv,
          rows, stage, acc, *sems):
    cid = lax.axis_index("c")
    sid = lax.axis_index("s")
    pltpu.async_copy(src_hbm.at[pl.ds(sid * k, k)], srcv, sems[0])
    pltpu.async_copy(dst_hbm.at[pl.ds(sid * k, k)], dstv, sems[1])
    pltpu.async_copy(zero_hbm.at[pl.ds(sid * RPT, RPT)],
                     acc.at[pl.ds(sid * RPT, RPT)], sems[2])
    stg = stage.at[pl.ds(sid * SPT, SPT)]

    @pl.when(cid == 0)
    def _sa():
      pltpu.async_copy(hsa_hbm.at[pl.ds(sid * SPT, SPT)], stg, sems[3])

    @pl.when(cid == 1)
    def _sb():
      pltpu.async_copy(hsb_hbm.at[pl.ds(sid * SPT, SPT)], stg, sems[3])

    pltpu.make_async_copy(src_hbm.at[pl.ds(sid * k, k)], srcv,
                          sems[0]).wait()
    pltpu.make_async_copy(dst_hbm.at[pl.ds(sid * k, k)], dstv,
                          sems[1]).wait()
    pltpu.make_async_copy(zero_hbm.at[pl.ds(sid * RPT, RPT)],
                          acc.at[pl.ds(sid * RPT, RPT)], sems[2]).wait()
    pltpu.make_async_copy(hsa_hbm.at[pl.ds(sid * SPT, SPT)], stg,
                          sems[3]).wait()
    plsc.subcore_barrier()
    _edge_loop(stage, acc, srcv, dstv, rows, sems, 2, k)
    plsc.subcore_barrier()
    pltpu.sync_copy(acc.at[pl.ds(sid * RPT, RPT)],
                    out_hbm.at[cid, pl.ds(sid * RPT, RPT)])

  return agg


@functools.lru_cache(maxsize=None)
def _make_agg_byedge():
  @functools.partial(
      pl.kernel,
      out_type=jax.ShapeDtypeStruct((NC, ACC_ROWS, DA), jnp.float32),
      mesh=_get_mesh(),
      scratch_types=[
          pltpu.VMEM((K, CHUNK), jnp.int32),
          pltpu.VMEM((K, CHUNK), jnp.int32),
          pltpu.VMEM((NBUF, CHUNK, DA), jnp.float32),
          pltpu.VMEM_SHARED((N, DA), jnp.float32),
          pltpu.VMEM_SHARED((ACC_ROWS, DA), jnp.float32),
      ] + [pltpu.SemaphoreType.DMA] * NBUF,
      compiler_params=pltpu.CompilerParams(use_tc_tiling_on_sc=False),
  )
  def agg(hs_hbm, src_hbm, dst_hbm, zero_hbm, out_hbm, srcv, dstv, rows,
          stage, acc, *sems):
    cid = lax.axis_index("c")
    sid = lax.axis_index("s")
    wid = sid * NC + cid
    pltpu.async_copy(src_hbm.at[pl.ds(wid * K, K)], srcv, sems[0])
    pltpu.async_copy(dst_hbm.at[pl.ds(wid * K, K)], dstv, sems[1])
    pltpu.async_copy(zero_hbm.at[pl.ds(sid * RPT, RPT)],
                     acc.at[pl.ds(sid * RPT, RPT)], sems[2])
    pltpu.async_copy(hs_hbm.at[pl.ds(sid * SPT, SPT)],
                     stage.at[pl.ds(sid * SPT, SPT)], sems[3])
    pltpu.make_async_copy(src_hbm.at[pl.ds(wid * K, K)], srcv,
                          sems[0]).wait()
    pltpu.make_async_copy(dst_hbm.at[pl.ds(wid * K, K)], dstv,
                          sems[1]).wait()
    pltpu.make_async_copy(zero_hbm.at[pl.ds(sid * RPT, RPT)],
                          acc.at[pl.ds(sid * RPT, RPT)], sems[2]).wait()
    pltpu.make_async_copy(hs_hbm.at[pl.ds(sid * SPT, SPT)],
                          stage.at[pl.ds(sid * SPT, SPT)], sems[3]).wait()
    plsc.subcore_barrier()
    _edge_loop(stage, acc, srcv, dstv, rows, sems, NBUF, K)
    plsc.subcore_barrier()
    pltpu.sync_copy(acc.at[pl.ds(sid * RPT, RPT)],
                    out_hbm.at[cid, pl.ds(sid * RPT, RPT)])

  return agg


@functools.lru_cache(maxsize=None)
def _make_deg():
  @functools.partial(
      pl.kernel,
      out_type=jax.ShapeDtypeStruct((NC, ACC_ROWS, 8), jnp.float32),
      mesh=_get_mesh(),
      scratch_types=[
          pltpu.VMEM((K, CHUNK), jnp.int32),
          pltpu.VMEM((CHUNK, 8), jnp.float32),
          pltpu.VMEM_SHARED((ACC_ROWS, 8), jnp.float32),
      ],
      compiler_params=pltpu.CompilerParams(use_tc_tiling_on_sc=False),
  )
  def deg(dst_hbm, ones_hbm, zero_hbm, out_hbm, dstv, ones_v, acc):
    cid = lax.axis_index("c")
    sid = lax.axis_index("s")
    wid = sid * NC + cid
    pltpu.sync_copy(dst_hbm.at[pl.ds(wid * K, K)], dstv)
    pltpu.sync_copy(ones_hbm, ones_v)
    pltpu.sync_copy(zero_hbm.at[pl.ds(sid * RPT, RPT)],
                    acc.at[pl.ds(sid * RPT, RPT)])
    plsc.subcore_barrier()

    def body(g, carry):
      pltpu.sync_copy(ones_v, acc.at[dstv.at[g]], add=True)
      return carry

    lax.fori_loop(0, K, body, 0)
    plsc.subcore_barrier()
    pltpu.sync_copy(acc.at[pl.ds(sid * RPT, RPT)],
                    out_hbm.at[cid, pl.ds(sid * RPT, RPT)])

  return deg


BR = 2000
_GRID = (N // BR,)


def _dinv(p0_ref, p1_ref):
  deg = p0_ref[0, :, 0:1] + p1_ref[0, :, 0:1] + 1.0
  return lax.rsqrt(deg)


def _tc1_body(x_ref, w1_ref, p0_ref, p1_ref, hsa_ref, hsb_ref):
  h = jnp.dot(x_ref[...], w1_ref[...], preferred_element_type=jnp.float32)
  hs = h * _dinv(p0_ref, p1_ref)
  hsa_ref[...] = hs[:, :DA]
  hsb_ref[...] = hs[:, DA:]


def _tc2_body(qa_ref, qb_ref, hsa_ref, hsb_ref, p0_ref,
              p1_ref, b1_ref, w2_ref, out_ref):
  dinv = _dinv(p0_ref, p1_ref)
  za = (qa_ref[0] + hsa_ref[...])
  zb = (qb_ref[0] + hsb_ref[...])
  z = jnp.concatenate([za, zb], axis=1) * dinv + b1_ref[...]
  z = jnp.maximum(z, 0.0)
  h2 = jnp.dot(z, w2_ref[...], preferred_element_type=jnp.float32)
  out_ref[...] = h2 * dinv


def _tc3_body(r0_ref, r1_ref, hs2_ref, p0_ref, p1_ref, b2_ref, out_ref):
  dinv = _dinv(p0_ref, p1_ref)
  out_ref[...] = (r0_ref[0] + r1_ref[0] + hs2_ref[...]) * dinv + b2_ref[...]


def _pspec(minor):
  return [
      pl.BlockSpec((1, BR, minor), lambda i: (0, i, 0)),
      pl.BlockSpec((1, BR, minor), lambda i: (1, i, 0)),
  ]


def kernel(X, edge_index, W1, b1, W2, b2):
  ei = edge_index.astype(jnp.int32)
  src, dst = ei[0], ei[1]
  pad = EP - E
  srcp = jnp.concatenate([src, jnp.zeros((pad,), jnp.int32)])
  dstp = jnp.concatenate([dst, jnp.full((pad,), N, jnp.int32)])
  src2d = srcp.reshape(ROWS2D, CHUNK)
  dst2d = dstp.reshape(ROWS2D, CHUNK)

  ones8 = jnp.ones((CHUNK, 8), jnp.float32)
  z8 = jnp.zeros((ACC_ROWS, 8), jnp.float32)
  zo = jnp.zeros((ACC_ROWS, DA), jnp.float32)

  degp = _make_deg()(dst2d, ones8, z8)

  hs1a, hs1b = pl.pallas_call(
      _tc1_body,
      grid=_GRID,
      in_specs=[
          pl.BlockSpec((BR, D_IN), lambda i: (i, 0)),
          pl.BlockSpec((D_IN, D_HID), lambda i: (0, 0)),
      ] + _pspec(8),
      out_specs=[
          pl.BlockSpec((BR, DA), lambda i: (i, 0)),
          pl.BlockSpec((BR, DA), lambda i: (i, 0)),
      ],
      out_shape=[
          jax.ShapeDtypeStruct((N, DA), jnp.float32),
          jax.ShapeDtypeStruct((N, DA), jnp.float32),
      ],
  )(X, W1, degp, degp)

  qab = _make_agg_bycol()(hs1a, hs1b, src2d, dst2d, zo)

  hs2 = pl.pallas_call(
      _tc2_body,
      grid=_GRID,
      in_specs=_pspec(DA) + [
          pl.BlockSpec((BR, DA), lambda i: (i, 0)),
          pl.BlockSpec((BR, DA), lambda i: (i, 0)),
      ] + _pspec(8) + [
          pl.BlockSpec((1, D_HID), lambda i: (0, 0)),
          pl.BlockSpec((D_HID, D_OUT), lambda i: (0, 0)),
      ],
      out_specs=pl.BlockSpec((BR, D_OUT), lambda i: (i, 0)),
      out_shape=jax.ShapeDtypeStruct((N, D_OUT), jnp.float32),
  )(qab, qab, hs1a, hs1b, degp, degp, b1.reshape(1, D_HID), W2)

  r = _make_agg_byedge()(hs2, src2d, dst2d, zo)

  out = pl.pallas_call(
      _tc3_body,
      grid=_GRID,
      in_specs=_pspec(D_OUT) + [
          pl.BlockSpec((BR, D_OUT), lambda i: (i, 0)),
      ] + _pspec(8) + [
          pl.BlockSpec((1, D_OUT), lambda i: (0, 0)),
      ],
      out_specs=pl.BlockSpec((BR, D_OUT), lambda i: (i, 0)),
      out_shape=jax.ShapeDtypeStruct((N, D_OUT), jnp.float32),
  )(r, r, hs2, degp, degp, b2.reshape(1, D_OUT))

  return out
